# Initial kernel scaffold; baseline (speedup 1.0000x reference)
import jax
import jax.numpy as jnp
from jax.experimental import pallas as pl


def kernel(data, species, W1, b1, W2, b2, W3, b3, W4, b4):
    raise NotImplementedError("write your pallas kernel here")


# speedup vs baseline: 1.2745x; 1.2745x over previous
import jax
import jax.numpy as jnp
from jax.experimental import pallas as pl
from jax.experimental.pallas import tpu as pltpu


def _celu(x, alpha=0.1):
    return jnp.where(x > 0, x, alpha * (jnp.exp(x / alpha) - 1.0))


def _mlp_body(ss_ref, ord_ref, x_ref, w1_ref, b1_ref, w2_ref, b2_ref,
              w3_ref, b3_ref, w4_ref, out_ref):
    a = pl.program_id(1)
    x = x_ref[...]
    x = jnp.where(jnp.isnan(x), jnp.zeros_like(x), x)
    h = _celu(jnp.dot(x, w1_ref[0], preferred_element_type=jnp.float32)
              + b1_ref[0])
    h = _celu(jnp.dot(h, w2_ref[0], preferred_element_type=jnp.float32)
              + b2_ref[0])
    h = _celu(jnp.dot(h, w3_ref[0], preferred_element_type=jnp.float32)
              + b3_ref[0])
    y = jnp.sum(h * w4_ref[0], axis=1)

    @pl.when(a == 0)
    def _():
        out_ref[0, 0, :] = y

    @pl.when(a > 0)
    def _():
        out_ref[0, 0, :] += y


def kernel(data, species, W1, b1, W2, b2, W3, b3, W4, b4):
    B, A, D = data.shape
    E, _, H1 = W1.shape
    H2 = W2.shape[2]
    H3 = W3.shape[2]
    LANE = 128

    species = species.astype(jnp.int32)
    order = jnp.argsort(species).astype(jnp.int32)
    ss = jnp.sort(species).astype(jnp.int32)

    W3p = jnp.pad(W3, ((0, 0), (0, 0), (0, LANE - H3)))
    b3p = jnp.pad(b3, ((0, 0), (0, LANE - H3))).reshape(E, 1, LANE)
    w4p = jnp.pad(W4[:, :, 0], ((0, 0), (0, LANE - H3))).reshape(E, 1, LANE)
    b1r = b1.reshape(E, 1, H1)
    b2r = b2.reshape(E, 1, H2)

    BB = 512
    nb = B // BB
    data2 = data.reshape(B, A * D)

    grid = (nb, A)

    def x_map(i, a, ss_ref, ord_ref):
        return (i, ord_ref[a])

    def w_map(i, a, ss_ref, ord_ref):
        return (ss_ref[a], 0, 0)

    def out_map(i, a, ss_ref, ord_ref):
        return (i, 0, 0)

    out3 = pl.pallas_call(
        _mlp_body,
        grid_spec=pltpu.PrefetchScalarGridSpec(
            num_scalar_prefetch=2,
            grid=grid,
            in_specs=[
                pl.BlockSpec((BB, D), x_map),
                pl.BlockSpec((1, D, H1), w_map),
                pl.BlockSpec((1, 1, H1), w_map),
                pl.BlockSpec((1, H1, H2), w_map),
                pl.BlockSpec((1, 1, H2), w_map),
                pl.BlockSpec((1, H2, LANE), w_map),
                pl.BlockSpec((1, 1, LANE), w_map),
                pl.BlockSpec((1, 1, LANE), w_map),
            ],
            out_specs=pl.BlockSpec((1, 1, BB), out_map),
        ),
        out_shape=jax.ShapeDtypeStruct((nb, 1, BB), jnp.float32),
    )(ss, order, data2, W1, b1r, W2, b2r, W3p, b3p, w4p)

    out = out3.reshape(B)
    out = out + jnp.sum(b4[species, 0])
    return out


# speedup vs baseline: 1.3096x; 1.0275x over previous
import jax
import jax.numpy as jnp
from jax.experimental import pallas as pl
from jax.experimental.pallas import tpu as pltpu


def _celu(x, alpha=0.1):
    return jnp.where(x > 0, x, alpha * (jnp.exp(x / alpha) - 1.0))


def _mlp_body(ss_ref, ord_ref, x_ref, w1_ref, b1_ref, w2_ref, b2_ref,
              w3_ref, b3_ref, w4_ref, out_ref):
    a = pl.program_id(1)
    x = x_ref[...]
    x = jnp.where(jnp.isnan(x), jnp.zeros_like(x), x)
    h = _celu(jnp.dot(x, w1_ref[0], preferred_element_type=jnp.float32)
              + b1_ref[0])
    h = _celu(jnp.dot(h, w2_ref[0], preferred_element_type=jnp.float32)
              + b2_ref[0])
    h = _celu(jnp.dot(h, w3_ref[0], preferred_element_type=jnp.float32)
              + b3_ref[0])
    y = jnp.dot(h, w4_ref[0], preferred_element_type=jnp.float32)

    @pl.when(a == 0)
    def _():
        out_ref[0] = y

    @pl.when(a > 0)
    def _():
        out_ref[0] += y


def kernel(data, species, W1, b1, W2, b2, W3, b3, W4, b4):
    B, A, D = data.shape
    E, _, H1 = W1.shape
    H2 = W2.shape[2]
    H3 = W3.shape[2]
    LANE = 128

    species = species.astype(jnp.int32)
    order = jnp.argsort(species).astype(jnp.int32)
    ss = jnp.sort(species).astype(jnp.int32)

    W3p = jnp.pad(W3, ((0, 0), (0, 0), (0, LANE - H3)))
    b3p = jnp.pad(b3, ((0, 0), (0, LANE - H3))).reshape(E, 1, LANE)
    w4p = jnp.pad(W4, ((0, 0), (0, LANE - H3), (0, LANE - 1)))
    b1r = b1.reshape(E, 1, H1)
    b2r = b2.reshape(E, 1, H2)

    BB = 512
    nb = B // BB
    data2 = data.reshape(B, A * D)

    grid = (nb, A)

    def x_map(i, a, ss_ref, ord_ref):
        return (i, ord_ref[a])

    def w_map(i, a, ss_ref, ord_ref):
        return (ss_ref[a], 0, 0)

    def out_map(i, a, ss_ref, ord_ref):
        return (i, 0, 0)

    out3 = pl.pallas_call(
        _mlp_body,
        grid_spec=pltpu.PrefetchScalarGridSpec(
            num_scalar_prefetch=2,
            grid=grid,
            in_specs=[
                pl.BlockSpec((BB, D), x_map),
                pl.BlockSpec((1, D, H1), w_map),
                pl.BlockSpec((1, 1, H1), w_map),
                pl.BlockSpec((1, H1, H2), w_map),
                pl.BlockSpec((1, 1, H2), w_map),
                pl.BlockSpec((1, H2, LANE), w_map),
                pl.BlockSpec((1, 1, LANE), w_map),
                pl.BlockSpec((1, LANE, LANE), w_map),
            ],
            out_specs=pl.BlockSpec((1, BB, LANE), out_map),
        ),
        out_shape=jax.ShapeDtypeStruct((nb, BB, LANE), jnp.float32),
    )(ss, order, data2, W1, b1r, W2, b2r, W3p, b3p, w4p)

    out = out3[:, :, 0].reshape(B)
    out = out + jnp.sum(b4[species, 0])
    return out


# speedup vs baseline: 2.1299x; 1.6264x over previous
import jax
import jax.numpy as jnp
from jax.experimental import pallas as pl
from jax.experimental.pallas import tpu as pltpu


def _celu(x, alpha=0.1):
    return jnp.where(x > 0, x, alpha * (jnp.exp(x / alpha) - 1.0))


def _make_body(A, BB, nsteps):
    def _mlp_body(ss_ref, ord_ref, x_hbm, w1_ref, b1_ref, w2_ref, b2_ref,
                  w3_ref, b3_ref, w4_ref, out_ref, xbuf, sem):
        i = pl.program_id(0)
        a = pl.program_id(1)
        t = i * A + a

        def x_copy(step):
            ii = step // A
            atom = ord_ref[step % A]
            slot = jax.lax.rem(step, 2)
            return pltpu.make_async_copy(
                x_hbm.at[pl.ds(ii * BB, BB), atom],
                xbuf.at[slot],
                sem.at[slot],
            )

        @pl.when(t == 0)
        def _():
            x_copy(0).start()

        @pl.when(t + 1 < nsteps)
        def _():
            x_copy(t + 1).start()

        x_copy(t).wait()
        x = xbuf[jax.lax.rem(t, 2)]
        x = jnp.where(jnp.isnan(x), jnp.zeros_like(x), x)
        h = _celu(jnp.dot(x, w1_ref[0], preferred_element_type=jnp.float32)
                  + b1_ref[0])
        h = _celu(jnp.dot(h, w2_ref[0], preferred_element_type=jnp.float32)
                  + b2_ref[0])
        h = _celu(jnp.dot(h, w3_ref[0], preferred_element_type=jnp.float32)
                  + b3_ref[0])
        y = jnp.dot(h, w4_ref[0], preferred_element_type=jnp.float32)

        @pl.when(a == 0)
        def _():
            out_ref[0] = y

        @pl.when(a > 0)
        def _():
            out_ref[0] += y

    return _mlp_body


def kernel(data, species, W1, b1, W2, b2, W3, b3, W4, b4):
    B, A, D = data.shape
    E, _, H1 = W1.shape
    H2 = W2.shape[2]
    H3 = W3.shape[2]
    LANE = 128

    species = species.astype(jnp.int32)
    order = jnp.argsort(species).astype(jnp.int32)
    ss = jnp.sort(species).astype(jnp.int32)

    W3p = jnp.pad(W3, ((0, 0), (0, 0), (0, LANE - H3)))
    b3p = jnp.pad(b3, ((0, 0), (0, LANE - H3))).reshape(E, 1, LANE)
    w4p = jnp.pad(W4, ((0, 0), (0, LANE - H3), (0, LANE - 1)))
    b1r = b1.reshape(E, 1, H1)
    b2r = b2.reshape(E, 1, H2)

    BB = 512
    nb = B // BB
    grid = (nb, A)
    nsteps = nb * A

    def w_map(i, a, ss_ref, ord_ref):
        return (ss_ref[a], 0, 0)

    def out_map(i, a, ss_ref, ord_ref):
        return (i, 0, 0)

    out3 = pl.pallas_call(
        _make_body(A, BB, nsteps),
        grid_spec=pltpu.PrefetchScalarGridSpec(
            num_scalar_prefetch=2,
            grid=grid,
            in_specs=[
                pl.BlockSpec(memory_space=pl.ANY),
                pl.BlockSpec((1, D, H1), w_map),
                pl.BlockSpec((1, 1, H1), w_map),
                pl.BlockSpec((1, H1, H2), w_map),
                pl.BlockSpec((1, 1, H2), w_map),
                pl.BlockSpec((1, H2, LANE), w_map),
                pl.BlockSpec((1, 1, LANE), w_map),
                pl.BlockSpec((1, LANE, LANE), w_map),
            ],
            out_specs=pl.BlockSpec((1, BB, LANE), out_map),
            scratch_shapes=[
                pltpu.VMEM((2, BB, D), jnp.float32),
                pltpu.SemaphoreType.DMA((2,)),
            ],
        ),
        out_shape=jax.ShapeDtypeStruct((nb, BB, LANE), jnp.float32),
    )(ss, order, data, W1, b1r, W2, b2r, W3p, b3p, w4p)

    out = out3[:, :, 0].reshape(B)
    out = out + jnp.sum(b4[species, 0])
    return out


# speedup vs baseline: 2.7144x; 1.2744x over previous
import jax
import jax.numpy as jnp
from jax.experimental import pallas as pl
from jax.experimental.pallas import tpu as pltpu


def _celu(x, alpha=0.1):
    return jnp.where(x > 0, x, alpha * (jnp.exp(x / alpha) - 1.0))


def _mlp_body(ss_ref, ord_ref, x_ref, w1_ref, b1_ref, w2_ref, b2_ref,
              w3_ref, b3_ref, w4_ref, out_ref):
    a = pl.program_id(1)
    x = x_ref[0]
    h = _celu(jnp.dot(x, w1_ref[0], preferred_element_type=jnp.float32)
              + b1_ref[0])
    h = _celu(jnp.dot(h, w2_ref[0], preferred_element_type=jnp.float32)
              + b2_ref[0])
    h = _celu(jnp.dot(h, w3_ref[0], preferred_element_type=jnp.float32)
              + b3_ref[0])
    y = jnp.dot(h, w4_ref[0], preferred_element_type=jnp.float32)

    @pl.when(a == 0)
    def _():
        out_ref[0] = y

    @pl.when(a > 0)
    def _():
        out_ref[0] += y


def kernel(data, species, W1, b1, W2, b2, W3, b3, W4, b4):
    B, A, D = data.shape
    E, _, H1 = W1.shape
    H2 = W2.shape[2]
    H3 = W3.shape[2]
    LANE = 128

    species = species.astype(jnp.int32)
    order = jnp.argsort(species).astype(jnp.int32)
    ss = jnp.sort(species).astype(jnp.int32)

    dataT = jnp.swapaxes(jnp.where(jnp.isnan(data), 0.0, data), 0, 1)

    W3p = jnp.pad(W3, ((0, 0), (0, 0), (0, LANE - H3)))
    b3p = jnp.pad(b3, ((0, 0), (0, LANE - H3))).reshape(E, 1, LANE)
    w4p = jnp.pad(W4, ((0, 0), (0, LANE - H3), (0, LANE - 1)))
    b1r = b1.reshape(E, 1, H1)
    b2r = b2.reshape(E, 1, H2)

    BB = 1024
    nb = B // BB
    grid = (nb, A)

    def x_map(i, a, ss_ref, ord_ref):
        return (ord_ref[a], i, 0)

    def w_map(i, a, ss_ref, ord_ref):
        return (ss_ref[a], 0, 0)

    def out_map(i, a, ss_ref, ord_ref):
        return (i, 0, 0)

    out3 = pl.pallas_call(
        _mlp_body,
        grid_spec=pltpu.PrefetchScalarGridSpec(
            num_scalar_prefetch=2,
            grid=grid,
            in_specs=[
                pl.BlockSpec((1, BB, D), x_map),
                pl.BlockSpec((1, D, H1), w_map),
                pl.BlockSpec((1, 1, H1), w_map),
                pl.BlockSpec((1, H1, H2), w_map),
                pl.BlockSpec((1, 1, H2), w_map),
                pl.BlockSpec((1, H2, LANE), w_map),
                pl.BlockSpec((1, 1, LANE), w_map),
                pl.BlockSpec((1, LANE, LANE), w_map),
            ],
            out_specs=pl.BlockSpec((1, BB, LANE), out_map),
        ),
        out_shape=jax.ShapeDtypeStruct((nb, BB, LANE), jnp.float32),
    )(ss, order, dataT, W1, b1r, W2, b2r, W3p, b3p, w4p)

    out = out3[:, :, 0].reshape(B)
    out = out + jnp.sum(b4[species, 0])
    return out


# speedup vs baseline: 3.2946x; 1.2138x over previous
import jax
import jax.numpy as jnp
from jax.experimental import pallas as pl
from jax.experimental.pallas import tpu as pltpu


def _celu(x, alpha=0.1):
    return jnp.where(x > 0, x, alpha * (jnp.exp(x / alpha) - 1.0))


def _mlp_body(ss_ref, ord_ref, x_ref, w1_ref, b1_ref, w2_ref, b2_ref,
              w3_ref, b3_ref, w4_ref, out_ref):
    a = pl.program_id(1)
    x = x_ref[0]
    h = _celu(jnp.dot(x, w1_ref[0], preferred_element_type=jnp.float32)
              + b1_ref[0])
    h = _celu(jnp.dot(h.astype(jnp.bfloat16), w2_ref[0],
                      preferred_element_type=jnp.float32) + b2_ref[0])
    h = _celu(jnp.dot(h.astype(jnp.bfloat16), w3_ref[0],
                      preferred_element_type=jnp.float32) + b3_ref[0])
    y = jnp.dot(h.astype(jnp.bfloat16), w4_ref[0],
                preferred_element_type=jnp.float32)

    @pl.when(a == 0)
    def _():
        out_ref[0] = y

    @pl.when(a > 0)
    def _():
        out_ref[0] += y


def kernel(data, species, W1, b1, W2, b2, W3, b3, W4, b4):
    B, A, D = data.shape
    E, _, H1 = W1.shape
    H2 = W2.shape[2]
    H3 = W3.shape[2]
    LANE = 128

    species = species.astype(jnp.int32)
    order = jnp.argsort(species).astype(jnp.int32)
    ss = jnp.sort(species).astype(jnp.int32)

    dataT = jnp.swapaxes(jnp.where(jnp.isnan(data), 0.0, data), 0, 1)
    dataT = dataT.astype(jnp.bfloat16)

    W3p = jnp.pad(W3, ((0, 0), (0, 0), (0, LANE - H3)))
    b3p = jnp.pad(b3, ((0, 0), (0, LANE - H3))).reshape(E, 1, LANE)
    w4p = jnp.pad(W4, ((0, 0), (0, LANE - H3), (0, LANE - 1)))
    b1r = b1.reshape(E, 1, H1)
    b2r = b2.reshape(E, 1, H2)

    W1 = W1.astype(jnp.bfloat16)
    W2 = W2.astype(jnp.bfloat16)
    W3p = W3p.astype(jnp.bfloat16)
    w4p = w4p.astype(jnp.bfloat16)

    BB = 1024
    nb = B // BB
    grid = (nb, A)

    def x_map(i, a, ss_ref, ord_ref):
        return (ord_ref[a], i, 0)

    def w_map(i, a, ss_ref, ord_ref):
        return (ss_ref[a], 0, 0)

    def out_map(i, a, ss_ref, ord_ref):
        return (i, 0, 0)

    out3 = pl.pallas_call(
        _mlp_body,
        grid_spec=pltpu.PrefetchScalarGridSpec(
            num_scalar_prefetch=2,
            grid=grid,
            in_specs=[
                pl.BlockSpec((1, BB, D), x_map),
                pl.BlockSpec((1, D, H1), w_map),
                pl.BlockSpec((1, 1, H1), w_map),
                pl.BlockSpec((1, H1, H2), w_map),
                pl.BlockSpec((1, 1, H2), w_map),
                pl.BlockSpec((1, H2, LANE), w_map),
                pl.BlockSpec((1, 1, LANE), w_map),
                pl.BlockSpec((1, LANE, LANE), w_map),
            ],
            out_specs=pl.BlockSpec((1, BB, LANE), out_map),
        ),
        out_shape=jax.ShapeDtypeStruct((nb, BB, LANE), jnp.float32),
    )(ss, order, dataT, W1, b1r, W2, b2r, W3p, b3p, w4p)

    out = out3[:, :, 0].reshape(B)
    out = out + jnp.sum(b4[species, 0])
    return out
